# baseline (speedup 1.0000x reference)
import functools

import jax
import jax.numpy as jnp
from jax import lax
from jax.experimental import pallas as pl
from jax.experimental.pallas import tpu as pltpu
from jax.experimental.pallas import tpu_sc as plsc

NC = 2
NS = 16
NW = NC * NS

_SC_PARAMS = pltpu.CompilerParams(use_tc_tiling_on_sc=False,
                                  needs_layout_passes=False)


def _degrees_call(E, NPAD, CH):
  EPW = E // NW
  NCH = EPW // CH
  ZS = NPAD // NS
  assert EPW % CH == 0 and NCH % 2 == 0
  mesh = plsc.VectorSubcoreMesh(
      core_axis_name="c", subcore_axis_name="s",
      num_cores=NC, num_subcores=NS)

  @functools.partial(
      pl.kernel,
      out_type=(jax.ShapeDtypeStruct((NC, NPAD), jnp.float32),
                jax.ShapeDtypeStruct((NC, NPAD), jnp.float32)),
      mesh=mesh,
      scratch_types=[
          pltpu.VMEM((2, CH), jnp.int32),
          pltpu.VMEM((2, CH), jnp.int32),
          pltpu.VMEM((CH,), jnp.float32),
          pltpu.VMEM_SHARED((NPAD,), jnp.float32),
          pltpu.VMEM_SHARED((NPAD,), jnp.float32),
          pltpu.SemaphoreType.DMA,
          pltpu.SemaphoreType.DMA,
      ],
      compiler_params=_SC_PARAMS,
  )
  def deg_kernel(src_h, dst_h, zeros_h, ones_h, dego_out, degi_out,
                 is_v, id_v, ones_v, dego_sh, degi_sh, sem_i, sem_s):
    c = lax.axis_index("c")
    s = lax.axis_index("s")
    wid = s * NC + c
    e0 = wid * EPW
    sl = pl.ds(s * ZS, ZS)
    pltpu.sync_copy(zeros_h.at[sl], dego_sh.at[sl])
    pltpu.sync_copy(zeros_h.at[sl], degi_sh.at[sl])
    pltpu.sync_copy(ones_h, ones_v)
    plsc.subcore_barrier()

    def load(g, b):
      pltpu.async_copy(src_h.at[pl.ds(e0 + g * CH, CH)], is_v.at[b], sem_i)
      pltpu.async_copy(dst_h.at[pl.ds(e0 + g * CH, CH)], id_v.at[b], sem_i)

    def wait_loads(b):
      pltpu.make_async_copy(src_h.at[pl.ds(0, CH)], is_v.at[b], sem_i).wait()
      pltpu.make_async_copy(dst_h.at[pl.ds(0, CH)], id_v.at[b], sem_i).wait()

    def scat(b):
      pltpu.async_copy(ones_v, dego_sh.at[is_v.at[b]], sem_s, add=True)
      pltpu.async_copy(ones_v, degi_sh.at[id_v.at[b]], sem_s, add=True)

    def wait_scat(b):
      pltpu.make_async_copy(ones_v, dego_sh.at[is_v.at[b]], sem_s).wait()
      pltpu.make_async_copy(ones_v, degi_sh.at[id_v.at[b]], sem_s).wait()

    load(0, 0)

    def step(g2, carry):
      for u in range(2):
        g = g2 * 2 + u
        b = u
        o = 1 - u
        wait_loads(b)

        @pl.when(g >= 1)
        def _():
          wait_scat(o)
        @pl.when(g + 1 < NCH)
        def _():
          load(g + 1, o)
        scat(b)
      return carry
    lax.fori_loop(0, NCH // 2, step, 0)
    wait_scat((NCH - 1) % 2)
    plsc.subcore_barrier()
    pltpu.sync_copy(dego_sh.at[sl], dego_out.at[c, sl])
    pltpu.sync_copy(degi_sh.at[sl], degi_out.at[c, sl])

  return deg_kernel


def _norm_call(NPAD, BL):
  grid = NPAD // BL

  def body(dgo_ref, dgi_ref, ft_ref, h_ref, nrm_ref):
    dego = dgo_ref[0:1, :] + dgo_ref[1:2, :]
    ns = lax.rsqrt(jnp.maximum(dego, 1.0))
    h_ref[...] = ft_ref[...] * ns
    degi = dgi_ref[0:1, :] + dgi_ref[1:2, :]
    nrm_ref[...] = lax.rsqrt(jnp.maximum(degi, 1.0))

  return pl.pallas_call(
      body,
      grid=(grid,),
      in_specs=[pl.BlockSpec((NC, BL), lambda i: (0, i)),
                pl.BlockSpec((NC, BL), lambda i: (0, i)),
                pl.BlockSpec((2, BL), lambda i: (0, i))],
      out_specs=(pl.BlockSpec((2, BL), lambda i: (0, i)),
                 pl.BlockSpec((1, BL), lambda i: (0, i))),
      out_shape=(jax.ShapeDtypeStruct((2, NPAD), jnp.float32),
                 jax.ShapeDtypeStruct((1, NPAD), jnp.float32)),
  )


def _agg_call(E, NPAD, CH, SUB):
  EPW = E // NW
  NCH = EPW // CH
  ZS = NPAD // NS
  assert EPW % CH == 0 and NCH % 4 == 0
  mesh = plsc.VectorSubcoreMesh(
      core_axis_name="c", subcore_axis_name="s",
      num_cores=NC, num_subcores=NS)

  @functools.partial(
      pl.kernel,
      out_type=jax.ShapeDtypeStruct((NC, 2, NPAD), jnp.float32),
      mesh=mesh,
      scratch_types=[
          pltpu.VMEM((4, CH), jnp.int32),
          pltpu.VMEM((4, CH), jnp.int32),
          pltpu.VMEM((2, CH), jnp.float32),
          pltpu.VMEM((2, CH), jnp.float32),
          pltpu.VMEM((SUB,), jnp.float32),
          pltpu.VMEM((SUB,), jnp.float32),
          pltpu.VMEM((SUB,), jnp.float32),
          pltpu.VMEM((SUB,), jnp.float32),
          pltpu.VMEM((SUB,), jnp.float32),
          pltpu.VMEM((SUB,), jnp.float32),
          pltpu.VMEM_SHARED((NPAD,), jnp.float32),
          pltpu.VMEM_SHARED((NPAD,), jnp.float32),
          pltpu.VMEM_SHARED((NPAD,), jnp.float32),
          pltpu.VMEM_SHARED((NPAD,), jnp.float32),
          pltpu.SemaphoreType.DMA,
          pltpu.SemaphoreType.DMA,
          pltpu.SemaphoreType.DMA,
      ],
      compiler_params=_SC_PARAMS,
  )
  def agg_kernel(src_h, dst_h, ft_h, dego_h, zeros_h, agg_out,
                 is_v, id_v, r0_v, r1_v,
                 d0_v, d1_v, f0_v, f1_v, hb0_v, hb1_v,
                 agg0_sh, agg1_sh, h0_sh, h1_sh, sem_i, sem_g, sem_s):
    c = lax.axis_index("c")
    s = lax.axis_index("s")
    wid = s * NC + c
    e0 = wid * EPW
    sl = pl.ds(s * ZS, ZS)
    pltpu.sync_copy(zeros_h.at[sl], agg0_sh.at[sl])
    pltpu.sync_copy(zeros_h.at[sl], agg1_sh.at[sl])
    for t in range(ZS // SUB):
      ssl = pl.ds(s * ZS + t * SUB, SUB)
      pltpu.sync_copy(dego_h.at[0, ssl], d0_v)
      pltpu.sync_copy(dego_h.at[1, ssl], d1_v)
      pltpu.sync_copy(ft_h.at[0, ssl], f0_v)
      pltpu.sync_copy(ft_h.at[1, ssl], f1_v)

      def nwt(j, carry):
        jj = pl.ds(j * 16, 16)
        x = jnp.maximum(d0_v[jj] + d1_v[jj], 1.0)
        i = plsc.bitcast(x, jnp.int32)
        y = plsc.bitcast(0x5F3759DF - lax.shift_right_logical(i, 1),
                         jnp.float32)
        y = y * (1.5 - 0.5 * x * y * y)
        y = y * (1.5 - 0.5 * x * y * y)
        y = y * (1.5 - 0.5 * x * y * y)
        hb0_v[jj] = f0_v[jj] * y
        hb1_v[jj] = f1_v[jj] * y
        return carry
      lax.fori_loop(0, SUB // 16, nwt, 0)
      pltpu.sync_copy(hb0_v, h0_sh.at[ssl])
      pltpu.sync_copy(hb1_v, h1_sh.at[ssl])
    plsc.subcore_barrier()

    def load(g, b4):
      pltpu.async_copy(src_h.at[pl.ds(e0 + g * CH, CH)], is_v.at[b4], sem_i)
      pltpu.async_copy(dst_h.at[pl.ds(e0 + g * CH, CH)], id_v.at[b4], sem_i)

    def wait_loads(b4):
      pltpu.make_async_copy(src_h.at[pl.ds(0, CH)], is_v.at[b4], sem_i).wait()
      pltpu.make_async_copy(dst_h.at[pl.ds(0, CH)], id_v.at[b4], sem_i).wait()

    def gath(b4, b2):
      pltpu.async_copy(h0_sh.at[is_v.at[b4]], r0_v.at[b2], sem_g)
      pltpu.async_copy(h1_sh.at[is_v.at[b4]], r1_v.at[b2], sem_g)

    def wait_gath(b4, b2):
      pltpu.make_async_copy(h0_sh.at[is_v.at[b4]], r0_v.at[b2], sem_g).wait()
      pltpu.make_async_copy(h1_sh.at[is_v.at[b4]], r1_v.at[b2], sem_g).wait()

    def scat(b4, b2):
      pltpu.async_copy(r0_v.at[b2], agg0_sh.at[id_v.at[b4]], sem_s, add=True)
      pltpu.async_copy(r1_v.at[b2], agg1_sh.at[id_v.at[b4]], sem_s, add=True)

    def wait_scat(b4, b2):
      pltpu.make_async_copy(r0_v.at[b2], agg0_sh.at[id_v.at[b4]], sem_s).wait()
      pltpu.make_async_copy(r1_v.at[b2], agg1_sh.at[id_v.at[b4]], sem_s).wait()

    load(0, 0)
    load(1, 1)

    def step(q, carry):
      for u in range(4):
        g = q * 4 + u
        b4 = u
        b2 = u % 2
        wait_loads(b4)

        @pl.when(g >= 2)
        def _():
          wait_scat((u + 2) % 4, b2)
        gath(b4, b2)

        @pl.when(g + 2 < NCH)
        def _():
          load(g + 2, (u + 2) % 4)
        wait_gath(b4, b2)
        scat(b4, b2)
      return carry
    lax.fori_loop(0, NCH // 4, step, 0)
    wait_scat((NCH - 2) % 4, (NCH - 2) % 2)
    wait_scat((NCH - 1) % 4, (NCH - 1) % 2)
    plsc.subcore_barrier()
    pltpu.sync_copy(agg0_sh.at[sl], agg_out.at[c, 0, sl])
    pltpu.sync_copy(agg1_sh.at[sl], agg_out.at[c, 1, sl])

  return agg_kernel


def _tanh_f32(x):
  xc = jnp.clip(x, -7.90531110763549805, 7.90531110763549805)
  x2 = xc * xc
  p = xc * (4.89352455891786e-03 + x2 * (6.37261928875436e-04 + x2 * (
      1.48572235717979e-05 + x2 * (5.12229709037114e-08 + x2 * (
          -8.60467152213735e-11 + x2 * (2.00018790482477e-13 + x2 * (
              -2.76076847742355e-16)))))))
  q = 4.89352518554385e-03 + x2 * (2.26843463243900e-03 + x2 * (
      1.18534705686654e-04 + x2 * 1.19825839466702e-06))
  return jnp.where(jnp.abs(x) < 0.0004, x, p / q)


def _tail_call(NPAD, BL, N, H):
  G = NPAD // BL

  def body(ap_ref, dgi_ref, WT_ref, bT_ref, W1T_ref, b1T_ref,
           W2T_ref, b2T_ref, W3T_ref, b3_ref, out_ref, acc_ref):
    i = pl.program_id(0)

    @pl.when(i == 0)
    def _init():
      acc_ref[...] = jnp.full((H, 1), -jnp.inf, jnp.float32)

    nrm = lax.rsqrt(jnp.maximum(dgi_ref[0:1, :] + dgi_ref[1:2, :], 1.0))
    agg0 = ap_ref[0:1, :] + ap_ref[2:3, :]
    agg1 = ap_ref[1:2, :] + ap_ref[3:4, :]
    aggT = jnp.concatenate([agg0, agg1], axis=0) * nrm
    z = lax.dot(WT_ref[...], aggT, preferred_element_type=jnp.float32,
                 precision=lax.Precision.HIGHEST)
    colid = lax.broadcasted_iota(jnp.int32, (1, BL), 1) + i * BL
    zm = jnp.where(colid < N, z, -jnp.inf)
    m = jnp.max(zm, axis=1, keepdims=True)
    acc_ref[...] = jnp.maximum(acc_ref[...], m)

    @pl.when(i == G - 1)
    def _tail():
      pooled = _tanh_f32(acc_ref[...] + bT_ref[...])
      x1 = lax.dot(W1T_ref[...], pooled,
                   preferred_element_type=jnp.float32,
                 precision=lax.Precision.HIGHEST) + b1T_ref[...]
      x1 = jnp.where(x1 > 0, x1, jnp.exp(jnp.minimum(x1, 0.0)) - 1.0)
      x2 = lax.dot(W2T_ref[...], x1,
                   preferred_element_type=jnp.float32,
                 precision=lax.Precision.HIGHEST) + b2T_ref[...]
      x2 = jnp.where(x2 > 0, x2, jnp.exp(jnp.minimum(x2, 0.0)) - 1.0)
      out_ref[...] = lax.dot(W3T_ref[...], x2,
                             preferred_element_type=jnp.float32,
                 precision=lax.Precision.HIGHEST) + b3_ref[...]

  full = lambda i: (0, 0)
  return pl.pallas_call(
      body,
      grid=(G,),
      in_specs=[pl.BlockSpec((4, BL), lambda i: (0, i)),
                pl.BlockSpec((NC, BL), lambda i: (0, i)),
                pl.BlockSpec((H, 2), full),
                pl.BlockSpec((H, 1), full),
                pl.BlockSpec((H, H), full),
                pl.BlockSpec((H, 1), full),
                pl.BlockSpec((32, H), full),
                pl.BlockSpec((32, 1), full),
                pl.BlockSpec((1, 32), full),
                pl.BlockSpec((1, 1), full)],
      out_specs=pl.BlockSpec((1, 1), full),
      out_shape=jax.ShapeDtypeStruct((1, 1), jnp.float32),
      scratch_shapes=[pltpu.VMEM((H, 1), jnp.float32)],
  )


def kernel(feats, edge_index, W, b, W1, b1, W2, b2, W3, b3):
  N = feats.shape[0]
  E = edge_index.shape[1]
  H = W.shape[1]
  CH1 = 20000
  CH3 = 5000
  BLN = 12800
  BLT = 4096
  NPAD = ((N + BLN - 1) // BLN) * BLN

  src = edge_index[0]
  dst = edge_index[1]
  zeros_n = jnp.zeros((NPAD,), jnp.float32)
  ones_c = jnp.ones((CH1,), jnp.float32)

  dego_p, degi_p = _degrees_call(E, NPAD, CH1)(src, dst, zeros_n, ones_c)

  featsT = jnp.pad(feats.T, ((0, 0), (0, NPAD - N)))
  aggp = _agg_call(E, NPAD, CH3, 1600)(src, dst, featsT, dego_p, zeros_n)

  out = _tail_call(NPAD, BLT, N, H)(
      aggp.reshape(4, NPAD), degi_p, W.T, b.reshape(H, 1),
      W1.T, b1.reshape(H, 1), W2.T, b2.reshape(32, 1),
      W3.T, b3.reshape(1, 1))
  return out


# speedup vs baseline: 1.0383x; 1.0383x over previous
import functools

import jax
import jax.numpy as jnp
from jax import lax
from jax.experimental import pallas as pl
from jax.experimental.pallas import tpu as pltpu
from jax.experimental.pallas import tpu_sc as plsc

NC = 2
NS = 16
NW = NC * NS

_SC_PARAMS = pltpu.CompilerParams(use_tc_tiling_on_sc=False,
                                  needs_layout_passes=False)


def _degrees_call(E, NPAD, CH):
  EPW = E // NW
  NCH = EPW // CH
  ZS = NPAD // NS
  assert EPW % CH == 0 and NCH % 2 == 0
  mesh = plsc.VectorSubcoreMesh(
      core_axis_name="c", subcore_axis_name="s",
      num_cores=NC, num_subcores=NS)

  @functools.partial(
      pl.kernel,
      out_type=(jax.ShapeDtypeStruct((NC, NPAD), jnp.float32),
                jax.ShapeDtypeStruct((NC, NPAD), jnp.float32)),
      mesh=mesh,
      scratch_types=[
          pltpu.VMEM((2, CH), jnp.int32),
          pltpu.VMEM((2, CH), jnp.int32),
          pltpu.VMEM((CH,), jnp.float32),
          pltpu.VMEM_SHARED((NPAD,), jnp.float32),
          pltpu.VMEM_SHARED((NPAD,), jnp.float32),
          pltpu.SemaphoreType.DMA,
          pltpu.SemaphoreType.DMA,
      ],
      compiler_params=_SC_PARAMS,
  )
  def deg_kernel(src_h, dst_h, zeros_h, ones_h, dego_out, degi_out,
                 is_v, id_v, ones_v, dego_sh, degi_sh, sem_i, sem_s):
    c = lax.axis_index("c")
    s = lax.axis_index("s")
    wid = s * NC + c
    e0 = wid * EPW
    sl = pl.ds(s * ZS, ZS)
    pltpu.sync_copy(zeros_h.at[sl], dego_sh.at[sl])
    pltpu.sync_copy(zeros_h.at[sl], degi_sh.at[sl])
    pltpu.sync_copy(ones_h, ones_v)
    plsc.subcore_barrier()

    def load(g, b):
      pltpu.async_copy(src_h.at[pl.ds(e0 + g * CH, CH)], is_v.at[b], sem_i)
      pltpu.async_copy(dst_h.at[pl.ds(e0 + g * CH, CH)], id_v.at[b], sem_i)

    def wait_loads(b):
      pltpu.make_async_copy(src_h.at[pl.ds(0, CH)], is_v.at[b], sem_i).wait()
      pltpu.make_async_copy(dst_h.at[pl.ds(0, CH)], id_v.at[b], sem_i).wait()

    def scat(b):
      pltpu.async_copy(ones_v, dego_sh.at[is_v.at[b]], sem_s, add=True)
      pltpu.async_copy(ones_v, degi_sh.at[id_v.at[b]], sem_s, add=True)

    def wait_scat(b):
      pltpu.make_async_copy(ones_v, dego_sh.at[is_v.at[b]], sem_s).wait()
      pltpu.make_async_copy(ones_v, degi_sh.at[id_v.at[b]], sem_s).wait()

    load(0, 0)

    def step(g2, carry):
      for u in range(2):
        g = g2 * 2 + u
        b = u
        o = 1 - u
        wait_loads(b)

        @pl.when(g >= 1)
        def _():
          wait_scat(o)
        @pl.when(g + 1 < NCH)
        def _():
          load(g + 1, o)
        scat(b)
      return carry
    lax.fori_loop(0, NCH // 2, step, 0)
    wait_scat((NCH - 1) % 2)
    plsc.subcore_barrier()
    pltpu.sync_copy(dego_sh.at[sl], dego_out.at[c, sl])
    pltpu.sync_copy(degi_sh.at[sl], degi_out.at[c, sl])

  return deg_kernel


def _norm_call(NPAD, BL):
  grid = NPAD // BL

  def body(dgo_ref, dgi_ref, ft_ref, h_ref, nrm_ref):
    dego = dgo_ref[0:1, :] + dgo_ref[1:2, :]
    ns = lax.rsqrt(jnp.maximum(dego, 1.0))
    h_ref[...] = ft_ref[...] * ns
    degi = dgi_ref[0:1, :] + dgi_ref[1:2, :]
    nrm_ref[...] = lax.rsqrt(jnp.maximum(degi, 1.0))

  return pl.pallas_call(
      body,
      grid=(grid,),
      in_specs=[pl.BlockSpec((NC, BL), lambda i: (0, i)),
                pl.BlockSpec((NC, BL), lambda i: (0, i)),
                pl.BlockSpec((2, BL), lambda i: (0, i))],
      out_specs=(pl.BlockSpec((2, BL), lambda i: (0, i)),
                 pl.BlockSpec((1, BL), lambda i: (0, i))),
      out_shape=(jax.ShapeDtypeStruct((2, NPAD), jnp.float32),
                 jax.ShapeDtypeStruct((1, NPAD), jnp.float32)),
  )


def _agg_call(E, NPAD, CH, SUB):
  EPW = E // NW
  NCH = EPW // CH
  ZS = NPAD // NS
  assert EPW % CH == 0 and NCH % 4 == 0
  mesh = plsc.VectorSubcoreMesh(
      core_axis_name="c", subcore_axis_name="s",
      num_cores=NC, num_subcores=NS)

  @functools.partial(
      pl.kernel,
      out_type=jax.ShapeDtypeStruct((NC, 2, NPAD), jnp.float32),
      mesh=mesh,
      scratch_types=[
          pltpu.VMEM((4, CH), jnp.int32),
          pltpu.VMEM((4, CH), jnp.int32),
          pltpu.VMEM((2, CH), jnp.float32),
          pltpu.VMEM((2, CH), jnp.float32),
          pltpu.VMEM((SUB,), jnp.float32),
          pltpu.VMEM((SUB,), jnp.float32),
          pltpu.VMEM((SUB,), jnp.float32),
          pltpu.VMEM((SUB,), jnp.float32),
          pltpu.VMEM((SUB,), jnp.float32),
          pltpu.VMEM((SUB,), jnp.float32),
          pltpu.VMEM_SHARED((NPAD,), jnp.float32),
          pltpu.VMEM_SHARED((NPAD,), jnp.float32),
          pltpu.VMEM_SHARED((NPAD,), jnp.float32),
          pltpu.VMEM_SHARED((NPAD,), jnp.float32),
          pltpu.SemaphoreType.DMA,
          pltpu.SemaphoreType.DMA,
          pltpu.SemaphoreType.DMA,
      ],
      compiler_params=_SC_PARAMS,
  )
  def agg_kernel(src_h, dst_h, ft_h, dego_h, zeros_h, agg_out,
                 is_v, id_v, r0_v, r1_v,
                 d0_v, d1_v, f0_v, f1_v, hb0_v, hb1_v,
                 agg0_sh, agg1_sh, h0_sh, h1_sh, sem_i, sem_g, sem_s):
    c = lax.axis_index("c")
    s = lax.axis_index("s")
    wid = s * NC + c
    e0 = wid * EPW
    sl = pl.ds(s * ZS, ZS)
    pltpu.sync_copy(zeros_h.at[sl], agg0_sh.at[sl])
    pltpu.sync_copy(zeros_h.at[sl], agg1_sh.at[sl])
    for t in range(ZS // SUB):
      ssl = pl.ds(s * ZS + t * SUB, SUB)
      pltpu.sync_copy(dego_h.at[0, ssl], d0_v)
      pltpu.sync_copy(dego_h.at[1, ssl], d1_v)
      pltpu.sync_copy(ft_h.at[0, ssl], f0_v)
      pltpu.sync_copy(ft_h.at[1, ssl], f1_v)

      def nwt(j, carry):
        jj = pl.ds(j * 16, 16)
        x = jnp.maximum(d0_v[jj] + d1_v[jj], 1.0)
        i = plsc.bitcast(x, jnp.int32)
        y = plsc.bitcast(0x5F3759DF - lax.shift_right_logical(i, 1),
                         jnp.float32)
        y = y * (1.5 - 0.5 * x * y * y)
        y = y * (1.5 - 0.5 * x * y * y)
        y = y * (1.5 - 0.5 * x * y * y)
        hb0_v[jj] = f0_v[jj] * y
        hb1_v[jj] = f1_v[jj] * y
        return carry
      lax.fori_loop(0, SUB // 16, nwt, 0)
      pltpu.sync_copy(hb0_v, h0_sh.at[ssl])
      pltpu.sync_copy(hb1_v, h1_sh.at[ssl])
    plsc.subcore_barrier()

    def load(g, b4):
      pltpu.async_copy(src_h.at[pl.ds(e0 + g * CH, CH)], is_v.at[b4], sem_i)
      pltpu.async_copy(dst_h.at[pl.ds(e0 + g * CH, CH)], id_v.at[b4], sem_i)

    def wait_loads(b4):
      pltpu.make_async_copy(src_h.at[pl.ds(0, CH)], is_v.at[b4], sem_i).wait()
      pltpu.make_async_copy(dst_h.at[pl.ds(0, CH)], id_v.at[b4], sem_i).wait()

    def gath(b4, b2):
      pltpu.async_copy(h0_sh.at[is_v.at[b4]], r0_v.at[b2], sem_g)
      pltpu.async_copy(h1_sh.at[is_v.at[b4]], r1_v.at[b2], sem_g)

    def wait_gath(b4, b2):
      pltpu.make_async_copy(h0_sh.at[is_v.at[b4]], r0_v.at[b2], sem_g).wait()
      pltpu.make_async_copy(h1_sh.at[is_v.at[b4]], r1_v.at[b2], sem_g).wait()

    def scat(b4, b2):
      pltpu.async_copy(r0_v.at[b2], agg0_sh.at[id_v.at[b4]], sem_s, add=True)
      pltpu.async_copy(r1_v.at[b2], agg1_sh.at[id_v.at[b4]], sem_s, add=True)

    def wait_scat(b4, b2):
      pltpu.make_async_copy(r0_v.at[b2], agg0_sh.at[id_v.at[b4]], sem_s).wait()
      pltpu.make_async_copy(r1_v.at[b2], agg1_sh.at[id_v.at[b4]], sem_s).wait()

    load(0, 0)
    load(1, 1)

    def step(q, carry):
      for u in range(4):
        g = q * 4 + u
        b4 = u
        b2 = u % 2
        wait_loads(b4)

        @pl.when(g >= 2)
        def _():
          wait_scat((u + 2) % 4, b2)
        gath(b4, b2)

        @pl.when(g + 2 < NCH)
        def _():
          load(g + 2, (u + 2) % 4)
        wait_gath(b4, b2)
        scat(b4, b2)
      return carry
    lax.fori_loop(0, NCH // 4, step, 0)
    wait_scat((NCH - 2) % 4, (NCH - 2) % 2)
    wait_scat((NCH - 1) % 4, (NCH - 1) % 2)
    plsc.subcore_barrier()
    pltpu.sync_copy(agg0_sh.at[sl], agg_out.at[c, 0, sl])
    pltpu.sync_copy(agg1_sh.at[sl], agg_out.at[c, 1, sl])

  return agg_kernel


def _tanh_f32(x):
  xc = jnp.clip(x, -7.90531110763549805, 7.90531110763549805)
  x2 = xc * xc
  p = xc * (4.89352455891786e-03 + x2 * (6.37261928875436e-04 + x2 * (
      1.48572235717979e-05 + x2 * (5.12229709037114e-08 + x2 * (
          -8.60467152213735e-11 + x2 * (2.00018790482477e-13 + x2 * (
              -2.76076847742355e-16)))))))
  q = 4.89352518554385e-03 + x2 * (2.26843463243900e-03 + x2 * (
      1.18534705686654e-04 + x2 * 1.19825839466702e-06))
  return jnp.where(jnp.abs(x) < 0.0004, x, p / q)


def _tail_call(NPAD, BL, N, H):
  G = NPAD // BL

  def body(ap_ref, dgi_ref, WT_ref, bT_ref, W1T_ref, b1T_ref,
           W2T_ref, b2T_ref, W3T_ref, b3_ref, out_ref, acc_ref):
    i = pl.program_id(0)

    @pl.when(i == 0)
    def _init():
      acc_ref[...] = jnp.full((H, 1), -jnp.inf, jnp.float32)

    nrm = lax.rsqrt(jnp.maximum(dgi_ref[0:1, :] + dgi_ref[1:2, :], 1.0))
    agg0 = (ap_ref[0:1, :] + ap_ref[2:3, :]) * nrm
    agg1 = (ap_ref[1:2, :] + ap_ref[3:4, :]) * nrm
    z = WT_ref[:, 0:1] * agg0 + WT_ref[:, 1:2] * agg1
    colid = lax.broadcasted_iota(jnp.int32, (1, BL), 1) + i * BL
    zm = jnp.where(colid < N, z, -jnp.inf)
    m = jnp.max(zm, axis=1, keepdims=True)
    acc_ref[...] = jnp.maximum(acc_ref[...], m)

    @pl.when(i == G - 1)
    def _tail():
      pooled = _tanh_f32(acc_ref[...] + bT_ref[...])
      x1 = lax.dot(W1T_ref[...], pooled,
                   preferred_element_type=jnp.float32,
                 precision=lax.Precision.HIGHEST) + b1T_ref[...]
      x1 = jnp.where(x1 > 0, x1, jnp.exp(jnp.minimum(x1, 0.0)) - 1.0)
      x2 = lax.dot(W2T_ref[...], x1,
                   preferred_element_type=jnp.float32,
                 precision=lax.Precision.HIGHEST) + b2T_ref[...]
      x2 = jnp.where(x2 > 0, x2, jnp.exp(jnp.minimum(x2, 0.0)) - 1.0)
      out_ref[...] = lax.dot(W3T_ref[...], x2,
                             preferred_element_type=jnp.float32,
                 precision=lax.Precision.HIGHEST) + b3_ref[...]

  full = lambda i: (0, 0)
  return pl.pallas_call(
      body,
      grid=(G,),
      in_specs=[pl.BlockSpec((4, BL), lambda i: (0, i)),
                pl.BlockSpec((NC, BL), lambda i: (0, i)),
                pl.BlockSpec((H, 2), full),
                pl.BlockSpec((H, 1), full),
                pl.BlockSpec((H, H), full),
                pl.BlockSpec((H, 1), full),
                pl.BlockSpec((32, H), full),
                pl.BlockSpec((32, 1), full),
                pl.BlockSpec((1, 32), full),
                pl.BlockSpec((1, 1), full)],
      out_specs=pl.BlockSpec((1, 1), full),
      out_shape=jax.ShapeDtypeStruct((1, 1), jnp.float32),
      scratch_shapes=[pltpu.VMEM((H, 1), jnp.float32)],
  )


def kernel(feats, edge_index, W, b, W1, b1, W2, b2, W3, b3):
  N = feats.shape[0]
  E = edge_index.shape[1]
  H = W.shape[1]
  CH1 = 10000
  CH3 = 5000
  BLN = 12800
  BLT = 2048
  NPAD = ((N + BLN - 1) // BLN) * BLN

  src = edge_index[0]
  dst = edge_index[1]
  zeros_n = jnp.zeros((NPAD,), jnp.float32)
  ones_c = jnp.ones((CH1,), jnp.float32)

  dego_p, degi_p = _degrees_call(E, NPAD, CH1)(src, dst, zeros_n, ones_c)

  featsT = jnp.pad(feats.T, ((0, 0), (0, NPAD - N)))
  aggp = _agg_call(E, NPAD, CH3, 1600)(src, dst, featsT, dego_p, zeros_n)

  out = _tail_call(NPAD, BLT, N, H)(
      aggp.reshape(4, NPAD), degi_p, W.T, b.reshape(H, 1),
      W1.T, b1.reshape(H, 1), W2.T, b2.reshape(32, 1),
      W3.T, b3.reshape(1, 1))
  return out
